# baseline (speedup 1.0000x reference)
import jax
import jax.numpy as jnp
from jax import lax
from jax.experimental import pallas as pl
from jax.experimental.pallas import tpu as pltpu
from jax.experimental.pallas import tpu_sc as plsc

_NUM_NODES = 1000000
_EMBED_DIM = 32
_BATCH = 16384
_LANES = 128
_CW = 512
_N_CHUNKS = 61
_RANGE = 31232
_OUT_ROWS = _BATCH + 16
_PIECE = 4096

_info = plsc.get_sparse_core_info()
_NC, _NS = _info.num_cores, _info.num_subcores
_NW = _NC * _NS


def _body(embT_hbm, idx_hbm, out_hbm, piece_v, hitn_v, hitp_v, chn_v, chp_v,
          chunk_v, stage_v, semc, sems):
    wid = lax.axis_index("s") * _NC + lax.axis_index("c")
    lo = wid * _RANGE
    hi = jnp.where(wid == _NW - 1, _NUM_NODES, lo + _RANGE)
    lo_al = pl.multiple_of(lo, _LANES)
    k16 = lax.iota(jnp.int32, 16)

    def fire(c):
        buf = lax.rem(c, 2)
        off = pl.multiple_of(lo_al + c * _CW, _LANES)
        pltpu.async_copy(
            embT_hbm.at[:, pl.ds(off, _CW)], chunk_v.at[buf], semc.at[buf]
        )

    fire(jnp.int32(0))
    fire(jnp.int32(1))

    def sel_piece(p, off):
        pltpu.sync_copy(idx_hbm.at[pl.ds(p * _PIECE, _PIECE)], piece_v)

        def sel(i, o):
            n16 = piece_v[pl.ds(i * 16, 16)]
            p16 = p * _PIECE + i * 16 + k16
            m = (n16 >= lo) & (n16 < hi)
            plsc.store_compressed(hitn_v.at[pl.ds(o, 16)], n16, mask=m)
            plsc.store_compressed(hitp_v.at[pl.ds(o, 16)], p16, mask=m)
            pc = plsc.all_reduce_population_count(m)
            return o + pc[0]

        return lax.fori_loop(0, _PIECE // 16, sel, off)

    nhits = lax.fori_loop(0, _BATCH // _PIECE, sel_piece, jnp.int32(0))

    def do_block(co, width, slot):
        def csel(g, off2):
            hn = hitn_v[pl.ds(g * 16, 16)]
            hp = hitp_v[pl.ds(g * 16, 16)]
            valid = (g * 16 + k16) < nhits
            m2 = valid & (hn >= co) & (hn < co + width)
            plsc.store_compressed(chn_v.at[pl.ds(off2, 16)], hn, mask=m2)
            plsc.store_compressed(chp_v.at[pl.ds(off2, 16)], hp, mask=m2)
            pc2 = plsc.all_reduce_population_count(m2)
            return off2 + pc2[0]

        nch = lax.fori_loop(0, (nhits + 15) // 16, csel, jnp.int32(0))

        def group(g, carry2):
            gbase = g * 16
            gvalid = gbase + k16 < nch
            nodes = plsc.load_gather(chn_v, [gbase + k16])
            poss = plsc.load_gather(chp_v, [gbase + k16])
            lanes = jnp.where(gvalid, nodes - co, 0)
            pos_pad = jnp.where(gvalid, poss, _BATCH + k16)
            for f in range(_EMBED_DIM):
                fvec = k16 * 0 + f
                vals = plsc.load_gather(chunk_v.at[slot], [fvec, lanes])
                plsc.store_scatter(stage_v, [k16, fvec], vals)
            pltpu.async_copy(stage_v, out_hbm.at[pos_pad], sems).wait()
            return carry2

        lax.fori_loop(0, (nch + 15) // 16, group, jnp.int32(0))

    def process_chunk(c, carry):
        buf = lax.rem(c, 2)
        pltpu.make_async_copy(
            embT_hbm.at[:, pl.ds(0, _CW)], chunk_v.at[buf], semc.at[buf]
        ).wait()
        do_block(lo_al + c * _CW, _CW, buf)

        @pl.when(c < _N_CHUNKS - 2)
        def _():
            fire(c + 2)
        return carry

    lax.fori_loop(0, _N_CHUNKS, process_chunk, jnp.int32(0))

    @pl.when(wid == _NW - 1)
    def _():
        t0 = jnp.int32(_NW * _RANGE)
        pltpu.sync_copy(
            embT_hbm.at[:, pl.ds(pl.multiple_of(t0, _LANES), _CW)],
            chunk_v.at[0],
        )
        do_block(t0, _CW, 0)
        t1 = jnp.int32(_NW * _RANGE + _CW)
        pltpu.sync_copy(
            embT_hbm.at[:, pl.ds(pl.multiple_of(t1, _LANES), _LANES)],
            chunk_v.at[1].at[:, pl.ds(0, _LANES)],
        )
        do_block(t1, _LANES, 1)


@jax.jit
def kernel(batch, emb):
    idx = batch.astype(jnp.int32)
    mesh = plsc.VectorSubcoreMesh(core_axis_name="c", subcore_axis_name="s")
    gather = pl.kernel(
        _body,
        out_type=jax.ShapeDtypeStruct((_OUT_ROWS, _LANES), jnp.float32),
        mesh=mesh,
        scratch_types=[
            pltpu.VMEM((_PIECE,), jnp.int32),
            pltpu.VMEM((_BATCH + 16,), jnp.int32),
            pltpu.VMEM((_BATCH + 16,), jnp.int32),
            pltpu.VMEM((_BATCH + 16,), jnp.int32),
            pltpu.VMEM((_BATCH + 16,), jnp.int32),
            pltpu.VMEM((2, _EMBED_DIM, _CW), jnp.float32),
            pltpu.VMEM((16, _LANES), jnp.float32),
            pltpu.SemaphoreType.DMA((2,)),
            pltpu.SemaphoreType.DMA,
        ],
        compiler_params=pltpu.CompilerParams(
            use_tc_tiling_on_sc=True, needs_layout_passes=False
        ),
    )
    out_wide = gather(emb.T, idx)
    return out_wide[:_BATCH, :_EMBED_DIM]


# speedup vs baseline: 2.0776x; 2.0776x over previous
import jax
import jax.numpy as jnp
from jax import lax
from jax.experimental import pallas as pl
from jax.experimental.pallas import tpu as pltpu
from jax.experimental.pallas import tpu_sc as plsc

_NUM_NODES = 1000000
_EMBED_DIM = 32
_BATCH = 16384
_LANES = 128

_info = plsc.get_sparse_core_info()
_NC, _NS = _info.num_cores, _info.num_subcores
_NW = _NC * _NS
_B_PER_W = _BATCH // _NW
_WAVE = 8
_N_WAVES = _B_PER_W // _WAVE
_RING = 3


def _gather_body(embT_hbm, idx_hbm, outT_hbm, idx_v, tiles_v, blk_v, sem_f):
    wid = lax.axis_index("s") * _NC + lax.axis_index("c")
    base = wid * _B_PER_W

    pltpu.sync_copy(idx_hbm.at[pl.ds(base, _B_PER_W)], idx_v.at[pl.ds(0, _B_PER_W)])

    def fire(w):
        buf = lax.rem(w, _RING)
        vecw = idx_v[pl.ds(w * _WAVE, 16)]
        for j in range(_WAVE):
            n = vecw[j]
            grp = pl.multiple_of((n // _LANES) * _LANES, _LANES)
            pltpu.async_copy(
                embT_hbm.at[:, pl.ds(grp, _LANES)],
                tiles_v.at[buf * _WAVE + j],
                sem_f,
            )

    fire(jnp.int32(0))
    fire(jnp.int32(1))

    k16 = lax.iota(jnp.int32, 16)
    jv = lax.rem(k16, _WAVE)
    fh = k16 // _WAVE

    def wave(w, carry):
        buf = lax.rem(w, _RING)
        @pl.when(w < _N_WAVES - 2)
        def _():
            fire(w + 2)

        for j in range(_WAVE):
            pltpu.make_async_copy(
                embT_hbm.at[:, pl.ds(0, _LANES)],
                tiles_v.at[buf * _WAVE + j],
                sem_f,
            ).wait()

        nvec = plsc.load_gather(idx_v, [w * _WAVE + jv])
        lanes = lax.rem(nvec, _LANES)
        slot = buf * _WAVE + jv
        col = w * _WAVE + jv
        for fg in range(_EMBED_DIM // 2):
            fvec = fh + 2 * fg
            vals = plsc.load_gather(tiles_v, [slot, fvec, lanes])
            plsc.store_scatter(blk_v, [fvec, col], vals)
        return carry

    lax.fori_loop(0, _N_WAVES, wave, jnp.int32(0))

    col0 = pl.multiple_of(base, _LANES)
    pltpu.sync_copy(blk_v, outT_hbm.at[:, pl.ds(col0, _B_PER_W)])


@jax.jit
def kernel(batch, emb):
    idx = batch.astype(jnp.int32)
    mesh = plsc.VectorSubcoreMesh(core_axis_name="c", subcore_axis_name="s")
    gather = pl.kernel(
        _gather_body,
        out_type=jax.ShapeDtypeStruct((_EMBED_DIM, _BATCH), jnp.float32),
        mesh=mesh,
        scratch_types=[
            pltpu.VMEM((_B_PER_W + _WAVE,), jnp.int32),
            pltpu.VMEM((_RING * _WAVE, _EMBED_DIM, _LANES), jnp.float32),
            pltpu.VMEM((_EMBED_DIM, _B_PER_W), jnp.float32),
            pltpu.SemaphoreType.DMA,
        ],
        compiler_params=pltpu.CompilerParams(
            use_tc_tiling_on_sc=True,
            needs_layout_passes=False,
            disable_bounds_checks=True,
            disable_semaphore_checks=True,
        ),
    )
    return gather(emb.T, idx).T


# speedup vs baseline: 2.0815x; 1.0019x over previous
import jax
import jax.numpy as jnp
from jax import lax
from jax.experimental import pallas as pl
from jax.experimental.pallas import tpu as pltpu
from jax.experimental.pallas import tpu_sc as plsc

_NUM_NODES = 1000000
_EMBED_DIM = 32
_BATCH = 16384
_LANES = 128

_info = plsc.get_sparse_core_info()
_NC, _NS = _info.num_cores, _info.num_subcores
_NW = _NC * _NS
_B_PER_W = _BATCH // _NW
_WAVE = 8
_N_WAVES = _B_PER_W // _WAVE
_RING = 3


def _gather_body(embT_hbm, idx_hbm, outT_hbm, idx_v, tiles_v, blk_v, sem_f):
    wid = lax.axis_index("s") * _NC + lax.axis_index("c")
    base = wid * _B_PER_W

    pltpu.sync_copy(idx_hbm.at[pl.ds(base, _B_PER_W)], idx_v.at[pl.ds(0, _B_PER_W)])

    def fire(w):
        buf = lax.rem(w, _RING)
        vecw = idx_v[pl.ds(w * _WAVE, 16)]
        for j in range(_WAVE):
            n = vecw[j]
            grp = pl.multiple_of((n // _LANES) * _LANES, _LANES)
            pltpu.async_copy(
                embT_hbm.at[:, pl.ds(grp, _LANES)],
                tiles_v.at[buf * _WAVE + j],
                sem_f,
            )

    fire(jnp.int32(0))
    fire(jnp.int32(1))

    k16 = lax.iota(jnp.int32, 16)
    jv = lax.rem(k16, _WAVE)
    fh = k16 // _WAVE

    def wave(w, carry):
        buf = lax.rem(w, _RING)
        @pl.when(w < _N_WAVES - 2)
        def _():
            fire(w + 2)

        for j in range(_WAVE):
            pltpu.make_async_copy(
                embT_hbm.at[:, pl.ds(0, _LANES)],
                tiles_v.at[buf * _WAVE + j],
                sem_f,
            ).wait()

        nvec = plsc.load_gather(idx_v, [w * _WAVE + jv])
        lanes = lax.rem(nvec, _LANES)
        slot = buf * _WAVE + jv
        col = w * _WAVE + jv
        for fg in range(_EMBED_DIM // 2):
            fvec = fh + 2 * fg
            vals = plsc.load_gather(tiles_v, [slot, fvec, lanes])
            plsc.store_scatter(blk_v, [fvec, col], vals)
        return carry

    lax.fori_loop(0, _N_WAVES, wave, jnp.int32(0))

    col0 = pl.multiple_of(base, _LANES)
    pltpu.sync_copy(blk_v, outT_hbm.at[:, pl.ds(col0, _B_PER_W)])


@jax.jit
def kernel(batch, emb):
    idx = batch.astype(jnp.int32)
    mesh = plsc.VectorSubcoreMesh(core_axis_name="c", subcore_axis_name="s")
    gather = pl.kernel(
        _gather_body,
        out_type=jax.ShapeDtypeStruct((_EMBED_DIM, _BATCH), jnp.float32),
        mesh=mesh,
        scratch_types=[
            pltpu.VMEM((_B_PER_W + _WAVE,), jnp.int32),
            pltpu.VMEM((_RING * _WAVE, _EMBED_DIM, _LANES), jnp.float32),
            pltpu.VMEM((_EMBED_DIM, _B_PER_W), jnp.float32),
            pltpu.SemaphoreType.DMA,
        ],
        compiler_params=pltpu.CompilerParams(
            use_tc_tiling_on_sc=True,
            needs_layout_passes=False,
            disable_bounds_checks=True,
            disable_semaphore_checks=True,
            skip_device_barrier=True,
        ),
    )
    return gather(emb.T, idx).T
